# baseline (speedup 1.0000x reference)
import functools
import math

import jax
import jax.numpy as jnp
from jax import lax
from jax.experimental import pallas as pl
from jax.experimental.pallas import tpu as pltpu
from jax.experimental.pallas import tpu_sc as plsc

D_MODEL = 1024
SCALE = math.sqrt(D_MODEL)
NC, NS, LANES = 2, 16, 16
NW = NC * NS
CHUNK = 16
NBUF = 4


def _embed_kernel(n_per_w, seq_len, table_hbm, x_hbm, out_hbm, idx_v, rows_v,
                  gsem, wsem):
    wid = lax.axis_index("s") * NC + lax.axis_index("c")
    base = wid * n_per_w
    w_per_row = seq_len // n_per_w
    b = wid // w_per_row
    off = (wid % w_per_row) * n_per_w
    pltpu.sync_copy(x_hbm.at[b, pl.ds(off, n_per_w)], idx_v)

    n_chunks = n_per_w // CHUNK

    def gather(c, nb):
        pltpu.async_copy(
            table_hbm.at[idx_v.at[pl.ds(c * CHUNK, CHUNK)]],
            rows_v.at[nb], gsem[nb])

    def wait_gather(nb):
        pltpu.make_async_copy(
            table_hbm.at[idx_v.at[pl.ds(0, CHUNK)]],
            rows_v.at[nb], gsem[nb]).wait()

    def writeback(c, nb):
        pltpu.async_copy(
            rows_v.at[nb], out_hbm.at[pl.ds(base + c * CHUNK, CHUNK)],
            wsem[nb])

    def wait_writeback(c, nb):
        pltpu.make_async_copy(
            rows_v.at[nb], out_hbm.at[pl.ds(base + c * CHUNK, CHUNK)],
            wsem[nb]).wait()

    for c in range(NBUF - 1):
        gather(c, c)
    for c in range(n_chunks):
        nb = c % NBUF
        if c + NBUF - 1 < n_chunks:
            if c >= 1:
                wait_writeback(c - 1, (c - 1) % NBUF)
            gather(c + NBUF - 1, (c + NBUF - 1) % NBUF)
        wait_gather(nb)

        @plsc.parallel_loop(0, CHUNK)
        def row_body(r):
            for v in range(D_MODEL // LANES):
                sl = pl.ds(v * LANES, LANES)
                rows_v[nb, r, sl] = rows_v[nb, r, sl] * SCALE

        writeback(c, nb)

    for c in range(max(0, n_chunks - NBUF), n_chunks):
        wait_writeback(c, c % NBUF)


@jax.jit
def kernel(x, table):
    B, L = x.shape
    n = B * L
    n_per_w = n // NW

    mesh = plsc.VectorSubcoreMesh(
        core_axis_name="c", subcore_axis_name="s", num_cores=NC, num_subcores=NS
    )
    out = pl.kernel(
        functools.partial(_embed_kernel, n_per_w, L),
        out_type=jax.ShapeDtypeStruct((n, D_MODEL), jnp.float32),
        mesh=mesh,
        scratch_types=[
            pltpu.VMEM((n_per_w,), jnp.int32),
            pltpu.VMEM((NBUF, CHUNK, D_MODEL), jnp.float32),
            [pltpu.SemaphoreType.DMA] * NBUF,
            [pltpu.SemaphoreType.DMA] * NBUF,
        ],
    )(table, x.astype(jnp.int32))
    return out.reshape(B, L, D_MODEL)


# speedup vs baseline: 1.0954x; 1.0954x over previous
import functools
import math

import jax
import jax.numpy as jnp
from jax import lax
from jax.experimental import pallas as pl
from jax.experimental.pallas import tpu as pltpu
from jax.experimental.pallas import tpu_sc as plsc

D_MODEL = 1024
SCALE = math.sqrt(D_MODEL)
NC, NS, LANES = 2, 16, 16
NW = NC * NS
CHUNK = 32
NBUF = 3


def _embed_kernel(n_per_w, seq_len, table_hbm, x_hbm, out_hbm, idx_v, rows_v,
                  gsem, wsem):
    wid = lax.axis_index("s") * NC + lax.axis_index("c")
    base = wid * n_per_w
    w_per_row = seq_len // n_per_w
    b = wid // w_per_row
    off = (wid % w_per_row) * n_per_w
    pltpu.sync_copy(x_hbm.at[b, pl.ds(off, n_per_w)], idx_v)

    n_chunks = n_per_w // CHUNK

    def gather(c, nb):
        pltpu.async_copy(
            table_hbm.at[idx_v.at[pl.ds(c * CHUNK, CHUNK)]],
            rows_v.at[nb], gsem[nb])

    def wait_gather(nb):
        pltpu.make_async_copy(
            table_hbm.at[idx_v.at[pl.ds(0, CHUNK)]],
            rows_v.at[nb], gsem[nb]).wait()

    def writeback(c, nb):
        pltpu.async_copy(
            rows_v.at[nb], out_hbm.at[pl.ds(base + c * CHUNK, CHUNK)],
            wsem[nb])

    def wait_writeback(c, nb):
        pltpu.make_async_copy(
            rows_v.at[nb], out_hbm.at[pl.ds(base + c * CHUNK, CHUNK)],
            wsem[nb]).wait()

    for c in range(NBUF - 1):
        gather(c, c)
    for c in range(n_chunks):
        nb = c % NBUF
        if c + NBUF - 1 < n_chunks:
            if c >= 1:
                wait_writeback(c - 1, (c - 1) % NBUF)
            gather(c + NBUF - 1, (c + NBUF - 1) % NBUF)
        wait_gather(nb)

        @plsc.parallel_loop(0, CHUNK)
        def row_body(r):
            for v in range(D_MODEL // LANES):
                sl = pl.ds(v * LANES, LANES)
                rows_v[nb, r, sl] = rows_v[nb, r, sl] * SCALE

        writeback(c, nb)

    for c in range(max(0, n_chunks - NBUF), n_chunks):
        wait_writeback(c, c % NBUF)


@jax.jit
def kernel(x, table):
    B, L = x.shape
    n = B * L
    n_per_w = n // NW

    mesh = plsc.VectorSubcoreMesh(
        core_axis_name="c", subcore_axis_name="s", num_cores=NC, num_subcores=NS
    )
    out = pl.kernel(
        functools.partial(_embed_kernel, n_per_w, L),
        out_type=jax.ShapeDtypeStruct((n, D_MODEL), jnp.float32),
        mesh=mesh,
        scratch_types=[
            pltpu.VMEM((n_per_w,), jnp.int32),
            pltpu.VMEM((NBUF, CHUNK, D_MODEL), jnp.float32),
            [pltpu.SemaphoreType.DMA] * NBUF,
            [pltpu.SemaphoreType.DMA] * NBUF,
        ],
    )(table, x.astype(jnp.int32))
    return out.reshape(B, L, D_MODEL)
